# baseline (speedup 1.0000x reference)
import jax
import jax.numpy as jnp
from jax.experimental import pallas as pl
from jax.experimental.pallas import tpu as pltpu

_NUM_TOKENS = 16384
_D_MODEL = 2048
_D_HALF = _D_MODEL // 2
_NUM_EXPERTS = 16
_BLOCK_T = 1024
_GRID = _NUM_TOKENS // _BLOCK_T


def _moe_gate_body(x1_ref, x2_ref, w_ref, ts_ref, ti_ref, loss_ref, acc_ref):
    step = pl.program_id(0)

    @pl.when(step == 0)
    def _init():
        acc_ref[...] = jnp.zeros_like(acc_ref)

    w = w_ref[...]
    logits = jax.lax.dot_general(
        x1_ref[...], w[:, :_D_HALF], (((1,), (1,)), ((), ())),
        preferred_element_type=jnp.float32)
    logits = logits + jax.lax.dot_general(
        x2_ref[...], w[:, _D_HALF:], (((1,), (1,)), ((), ())),
        preferred_element_type=jnp.float32)
    lt = logits.T

    m = jnp.max(lt, axis=0, keepdims=True)
    e = jnp.exp(lt - m)
    s = jnp.sum(e, axis=0, keepdims=True)
    scores = e / s

    acc_ref[...] += jnp.sum(scores, axis=1, keepdims=True)

    row = jax.lax.broadcasted_iota(jnp.int32, scores.shape, 0)
    v1 = jnp.max(scores, axis=0, keepdims=True)
    i1 = jnp.min(jnp.where(scores == v1, row, _NUM_EXPERTS),
                 axis=0, keepdims=True)
    masked = jnp.where(row == i1, -jnp.inf, scores)
    v2 = jnp.max(masked, axis=0, keepdims=True)
    i2 = jnp.min(jnp.where(masked == v2, row, _NUM_EXPERTS),
                 axis=0, keepdims=True)

    denom = v1 + v2
    ts_t = jnp.concatenate([v1 / denom, v2 / denom], axis=0)
    ti_t = jnp.concatenate([i1, i2], axis=0)
    ts_ref[...] = ts_t.T
    ti_ref[...] = ti_t.T

    @pl.when(step == _GRID - 1)
    def _fin():
        p = acc_ref[...] / _NUM_TOKENS
        loss_ref[0, 0] = jnp.sum(p * jnp.log(p + 1e-8))


def kernel(x, W):
    ts, ti, loss = pl.pallas_call(
        _moe_gate_body,
        grid=(_GRID,),
        in_specs=[
            pl.BlockSpec((_BLOCK_T, _D_HALF), lambda i: (i, 0)),
            pl.BlockSpec((_BLOCK_T, _D_HALF), lambda i: (i, 1)),
            pl.BlockSpec((_NUM_EXPERTS, _D_MODEL), lambda i: (0, 0)),
        ],
        out_specs=[
            pl.BlockSpec((_BLOCK_T, 2), lambda i: (i, 0)),
            pl.BlockSpec((_BLOCK_T, 2), lambda i: (i, 0)),
            pl.BlockSpec(memory_space=pltpu.SMEM, block_shape=(1, 1),
                         index_map=lambda i: (0, 0)),
        ],
        out_shape=[
            jax.ShapeDtypeStruct((_NUM_TOKENS, 2), jnp.float32),
            jax.ShapeDtypeStruct((_NUM_TOKENS, 2), jnp.int32),
            jax.ShapeDtypeStruct((1, 1), jnp.float32),
        ],
        scratch_shapes=[pltpu.VMEM((_NUM_EXPERTS, 1), jnp.float32)],
        compiler_params=pltpu.CompilerParams(
            dimension_semantics=("arbitrary",)),
    )(x, x, W)
    return ts, ti, loss.reshape(())


# speedup vs baseline: 1.0183x; 1.0183x over previous
import jax
import jax.numpy as jnp
from jax.experimental import pallas as pl
from jax.experimental.pallas import tpu as pltpu

_NUM_TOKENS = 16384
_D_MODEL = 2048
_D_HALF = _D_MODEL // 2
_NUM_EXPERTS = 16
_BLOCK_T = 2048
_GRID = _NUM_TOKENS // _BLOCK_T


def _moe_gate_body(x1_ref, x2_ref, w_ref, ts_ref, ti_ref, loss_ref, acc_ref):
    step = pl.program_id(0)

    @pl.when(step == 0)
    def _init():
        acc_ref[...] = jnp.zeros_like(acc_ref)

    w = w_ref[...]
    logits = jnp.concatenate([
        jax.lax.dot_general(
            x1_ref[...], w, (((1,), (1,)), ((), ())),
            preferred_element_type=jnp.float32),
        jax.lax.dot_general(
            x2_ref[...], w, (((1,), (1,)), ((), ())),
            preferred_element_type=jnp.float32),
    ], axis=0)
    lt = logits.T

    m = jnp.max(lt, axis=0, keepdims=True)
    e = jnp.exp(lt - m)
    s = jnp.sum(e, axis=0, keepdims=True)
    scores = e / s

    acc_ref[...] += jnp.sum(scores, axis=1, keepdims=True)

    row = jax.lax.broadcasted_iota(jnp.int32, scores.shape, 0)
    v1 = jnp.max(scores, axis=0, keepdims=True)
    i1 = jnp.min(jnp.where(scores == v1, row, _NUM_EXPERTS),
                 axis=0, keepdims=True)
    masked = jnp.where(row == i1, -jnp.inf, scores)
    v2 = jnp.max(masked, axis=0, keepdims=True)
    i2 = jnp.min(jnp.where(masked == v2, row, _NUM_EXPERTS),
                 axis=0, keepdims=True)

    denom = v1 + v2
    ts_t = jnp.concatenate([v1 / denom, v2 / denom], axis=0)
    ti_t = jnp.concatenate([i1, i2], axis=0)
    ts_ref[...] = ts_t.T
    ti_ref[...] = ti_t.T

    @pl.when(step == _GRID - 1)
    def _fin():
        p = acc_ref[...] / _NUM_TOKENS
        loss_ref[0, 0] = jnp.sum(p * jnp.log(p + 1e-8))


def kernel(x, W):
    ts, ti, loss = pl.pallas_call(
        _moe_gate_body,
        grid=(_GRID,),
        in_specs=[
            pl.BlockSpec((_BLOCK_T // 2, _D_MODEL), lambda i: (2 * i, 0)),
            pl.BlockSpec((_BLOCK_T // 2, _D_MODEL), lambda i: (2 * i + 1, 0)),
            pl.BlockSpec((_NUM_EXPERTS, _D_MODEL), lambda i: (0, 0)),
        ],
        out_specs=[
            pl.BlockSpec((_BLOCK_T, 2), lambda i: (i, 0)),
            pl.BlockSpec((_BLOCK_T, 2), lambda i: (i, 0)),
            pl.BlockSpec(memory_space=pltpu.SMEM, block_shape=(1, 1),
                         index_map=lambda i: (0, 0)),
        ],
        out_shape=[
            jax.ShapeDtypeStruct((_NUM_TOKENS, 2), jnp.float32),
            jax.ShapeDtypeStruct((_NUM_TOKENS, 2), jnp.int32),
            jax.ShapeDtypeStruct((1, 1), jnp.float32),
        ],
        scratch_shapes=[pltpu.VMEM((_NUM_EXPERTS, 1), jnp.float32)],
        compiler_params=pltpu.CompilerParams(
            dimension_semantics=("arbitrary",)),
    )(x, x, W)
    return ts, ti, loss.reshape(())


# speedup vs baseline: 1.3674x; 1.3428x over previous
import jax
import jax.numpy as jnp
from jax.experimental import pallas as pl
from jax.experimental.pallas import tpu as pltpu

_NUM_TOKENS = 16384
_D_MODEL = 2048
_NUM_EXPERTS = 16
_BLOCK_T = 2048
_GRID = _NUM_TOKENS // _BLOCK_T


def _moe_gate_body(x_ref, w_ref, ts_ref, ti_ref, loss_ref, acc_ref):
    step = pl.program_id(0)

    @pl.when(step == 0)
    def _init():
        acc_ref[...] = jnp.zeros_like(acc_ref)

    w = w_ref[...]
    logits = jax.lax.dot_general(
        x_ref[...], w, (((1,), (1,)), ((), ())),
        preferred_element_type=jnp.float32)
    lt = logits.T

    m = jnp.max(lt, axis=0, keepdims=True)
    e = jnp.exp(lt - m)
    s = jnp.sum(e, axis=0, keepdims=True)
    scores = e / s

    acc_ref[...] += jnp.sum(scores, axis=1, keepdims=True)

    row = jax.lax.broadcasted_iota(jnp.int32, scores.shape, 0)
    v1 = jnp.max(scores, axis=0, keepdims=True)
    i1 = jnp.min(jnp.where(scores == v1, row, _NUM_EXPERTS),
                 axis=0, keepdims=True)
    masked = jnp.where(row == i1, -jnp.inf, scores)
    v2 = jnp.max(masked, axis=0, keepdims=True)
    i2 = jnp.min(jnp.where(masked == v2, row, _NUM_EXPERTS),
                 axis=0, keepdims=True)

    denom = v1 + v2
    ts_ref[...] = jnp.concatenate([v1 / denom, v2 / denom], axis=0)
    ti_ref[...] = jnp.concatenate([i1, i2], axis=0)

    @pl.when(step == _GRID - 1)
    def _fin():
        p = acc_ref[...] / _NUM_TOKENS
        loss_ref[0, 0] = jnp.sum(p * jnp.log(p + 1e-8))


def kernel(x, W):
    ts_t, ti_t, loss = pl.pallas_call(
        _moe_gate_body,
        grid=(_GRID,),
        in_specs=[
            pl.BlockSpec((_BLOCK_T, _D_MODEL), lambda i: (i, 0)),
            pl.BlockSpec((_NUM_EXPERTS, _D_MODEL), lambda i: (0, 0)),
        ],
        out_specs=[
            pl.BlockSpec((2, _BLOCK_T), lambda i: (0, i)),
            pl.BlockSpec((2, _BLOCK_T), lambda i: (0, i)),
            pl.BlockSpec(memory_space=pltpu.SMEM, block_shape=(1, 1),
                         index_map=lambda i: (0, 0)),
        ],
        out_shape=[
            jax.ShapeDtypeStruct((2, _NUM_TOKENS), jnp.float32),
            jax.ShapeDtypeStruct((2, _NUM_TOKENS), jnp.int32),
            jax.ShapeDtypeStruct((1, 1), jnp.float32),
        ],
        scratch_shapes=[pltpu.VMEM((_NUM_EXPERTS, 1), jnp.float32)],
        compiler_params=pltpu.CompilerParams(
            dimension_semantics=("arbitrary",)),
    )(x, W)
    return ts_t.T, ti_t.T, loss.reshape(())


# speedup vs baseline: 1.4264x; 1.0432x over previous
import jax
import jax.numpy as jnp
from jax.experimental import pallas as pl
from jax.experimental.pallas import tpu as pltpu

_NUM_TOKENS = 16384
_D_MODEL = 2048
_NUM_EXPERTS = 16
_BLOCK_T = 1024
_GRID = _NUM_TOKENS // _BLOCK_T


def _moe_gate_body(x_ref, w_ref, ts_ref, ti_ref, loss_ref, acc_ref):
    step = pl.program_id(0)

    @pl.when(step == 0)
    def _init():
        acc_ref[...] = jnp.zeros_like(acc_ref)

    w = w_ref[...]
    logits = jax.lax.dot_general(
        x_ref[...], w, (((1,), (1,)), ((), ())),
        preferred_element_type=jnp.float32)
    lt = logits.T

    m = jnp.max(lt, axis=0, keepdims=True)
    e = jnp.exp(lt - m)
    s = jnp.sum(e, axis=0, keepdims=True)
    scores = e / s

    acc_ref[...] += jnp.sum(scores, axis=1, keepdims=True)

    row = jax.lax.broadcasted_iota(jnp.int32, scores.shape, 0)
    v1 = jnp.max(scores, axis=0, keepdims=True)
    i1 = jnp.min(jnp.where(scores == v1, row, _NUM_EXPERTS),
                 axis=0, keepdims=True)
    masked = jnp.where(row == i1, -jnp.inf, scores)
    v2 = jnp.max(masked, axis=0, keepdims=True)
    i2 = jnp.min(jnp.where(masked == v2, row, _NUM_EXPERTS),
                 axis=0, keepdims=True)

    denom = v1 + v2
    ts_ref[...] = jnp.concatenate([v1 / denom, v2 / denom], axis=0)
    ti_ref[...] = jnp.concatenate([i1, i2], axis=0)

    @pl.when(step == _GRID - 1)
    def _fin():
        p = acc_ref[...] / _NUM_TOKENS
        loss_ref[0, 0] = jnp.sum(p * jnp.log(p + 1e-8))


def kernel(x, W):
    ts_t, ti_t, loss = pl.pallas_call(
        _moe_gate_body,
        grid=(_GRID,),
        in_specs=[
            pl.BlockSpec((_BLOCK_T, _D_MODEL), lambda i: (i, 0)),
            pl.BlockSpec((_NUM_EXPERTS, _D_MODEL), lambda i: (0, 0)),
        ],
        out_specs=[
            pl.BlockSpec((2, _BLOCK_T), lambda i: (0, i)),
            pl.BlockSpec((2, _BLOCK_T), lambda i: (0, i)),
            pl.BlockSpec(memory_space=pltpu.SMEM, block_shape=(1, 1),
                         index_map=lambda i: (0, 0)),
        ],
        out_shape=[
            jax.ShapeDtypeStruct((2, _NUM_TOKENS), jnp.float32),
            jax.ShapeDtypeStruct((2, _NUM_TOKENS), jnp.int32),
            jax.ShapeDtypeStruct((1, 1), jnp.float32),
        ],
        scratch_shapes=[pltpu.VMEM((_NUM_EXPERTS, 1), jnp.float32)],
        compiler_params=pltpu.CompilerParams(
            dimension_semantics=("arbitrary",)),
    )(x, W)
    return ts_t.T, ti_t.T, loss.reshape(())
